# Initial kernel scaffold; baseline (speedup 1.0000x reference)
import jax
import jax.numpy as jnp
from jax.experimental import pallas as pl


def kernel(input_ids, tok_table, pos_table, type_table, gamma, beta):
    raise NotImplementedError("write your pallas kernel here")


# speedup vs baseline: 1.0476x; 1.0476x over previous
import functools

import jax
import jax.numpy as jnp
from jax import lax
from jax.experimental import pallas as pl
from jax.experimental.pallas import tpu as pltpu
from jax.experimental.pallas import tpu_sc as plsc

DIM = 64
LANES = 16
CH = 256
NW = 32
SEQ = 200
NTOK = 4096 * SEQ
TPW = NTOK // NW
NCH = TPW // CH
IDROWS = TPW // 128


def _sc_embed_ln(ids_hbm, tok_hbm, comb_hbm, gam_hbm, bet_hbm, out_hbm,
                 ids_v, rows0, rows1, ybuf, comb_v, gam_v, bet_v,
                 mn_v, rs_v, gsem0, gsem1, osem):
    wid = lax.axis_index("s") * 2 + lax.axis_index("c")
    base_tok = wid * TPW

    pltpu.sync_copy(ids_hbm.at[pl.ds(wid * IDROWS, IDROWS)], ids_v)
    pltpu.sync_copy(comb_hbm, comb_v)
    pltpu.sync_copy(gam_hbm, gam_v)
    pltpu.sync_copy(bet_hbm, bet_v)

    iota = lax.iota(jnp.int32, LANES)

    def fire_gather(c, rows, sem):
        for j in range(CH // 128):
            pltpu.async_copy(tok_hbm.at[ids_v.at[c * (CH // 128) + j]],
                             rows.at[pl.ds(j * 128, 128)], sem)

    def wait_gather(rows, sem):
        pltpu.make_async_copy(tok_hbm.at[pl.ds(0, CH)], rows, sem).wait()

    def fire_store(c, sem):
        pltpu.async_copy(ybuf, out_hbm.at[pl.ds(base_tok + c * CH, CH)], sem)

    def wait_store(sem):
        pltpu.make_async_copy(ybuf, out_hbm.at[pl.ds(0, CH)], sem).wait()

    half = jnp.float32(0.5)
    threehalf = jnp.float32(1.5)
    inv_dim = jnp.float32(1.0 / DIM)

    def pass1(c, rows):
        def group_body(g, _):
            row16 = g * LANES + iota
            loc = c * CH + row16
            ids16 = plsc.load_gather(ids_v, [loc // 128, loc % 128])
            flat = base_tok + loc
            l16 = flat % SEQ
            pos16 = jnp.where(ids16 == 0, 0, l16 + 1)

            def dbody(d, sq):
                s, q = sq
                dcol = jnp.full((LANES,), d, dtype=jnp.int32)
                tok = plsc.load_gather(rows, [row16, dcol])
                com = plsc.load_gather(comb_v, [pos16, dcol])
                v = tok + com
                plsc.store_scatter(rows, [row16, dcol], v)
                return s + v, q + v * v

            s, q = lax.fori_loop(
                0, DIM, dbody,
                (jnp.zeros((LANES,), jnp.float32),
                 jnp.zeros((LANES,), jnp.float32)),
                unroll=4)
            mean = s * inv_dim
            var = q * inv_dim - mean * mean
            x = var + jnp.float32(1e-12)
            i = lax.bitcast_convert_type(x, jnp.int32)
            i = jnp.int32(0x5F3759DF) - (i >> 1)
            y = lax.bitcast_convert_type(i, jnp.float32)
            y = y * (threehalf - half * x * y * y)
            y = y * (threehalf - half * x * y * y)
            y = y * (threehalf - half * x * y * y)
            mn_v[pl.ds(g * LANES, LANES)] = mean
            rs_v[pl.ds(g * LANES, LANES)] = y
            return 0

        lax.fori_loop(0, CH // LANES, group_body, 0)

    def pass2(rows):
        def group_body(g, _):
            row16 = g * LANES + iota
            mean = mn_v[pl.ds(g * LANES, LANES)]
            rstd = rs_v[pl.ds(g * LANES, LANES)]

            def dbody(d, _):
                dcol = jnp.full((LANES,), d, dtype=jnp.int32)
                a = rstd * gam_v[d, :]
                b = bet_v[d, :] - mean * a
                v = plsc.load_gather(rows, [row16, dcol])
                plsc.store_scatter(ybuf, [row16, dcol], v * a + b)
                return 0

            lax.fori_loop(0, DIM, dbody, 0, unroll=4)
            return 0

        lax.fori_loop(0, CH // LANES, group_body, 0)

    fire_gather(0, rows0, gsem0)

    def pair_body(p, _):
        c0 = p * 2
        c1 = c0 + 1
        wait_gather(rows0, gsem0)
        fire_gather(c1, rows1, gsem1)
        pass1(c0, rows0)

        @pl.when(p > 0)
        def _():
            wait_store(osem)

        pass2(rows0)
        fire_store(c0, osem)

        wait_gather(rows1, gsem1)

        @pl.when(p < NCH // 2 - 1)
        def _():
            fire_gather(c0 + 2, rows0, gsem0)

        pass1(c1, rows1)
        wait_store(osem)
        pass2(rows1)
        fire_store(c1, osem)
        return 0

    lax.fori_loop(0, NCH // 2, pair_body, 0)
    wait_store(osem)


@functools.partial(
    pl.kernel,
    mesh=plsc.VectorSubcoreMesh(core_axis_name="c", subcore_axis_name="s"),
    out_type=jax.ShapeDtypeStruct((NTOK, DIM), jnp.float32),
    compiler_params=pltpu.CompilerParams(use_tc_tiling_on_sc=False,
                                        needs_layout_passes=False),
    scratch_types=[
        pltpu.VMEM((IDROWS, 128), jnp.int32),
        pltpu.VMEM((CH, DIM), jnp.float32),
        pltpu.VMEM((CH, DIM), jnp.float32),
        pltpu.VMEM((CH, DIM), jnp.float32),
        pltpu.VMEM((SEQ + 1, DIM), jnp.float32),
        pltpu.VMEM((DIM, LANES), jnp.float32),
        pltpu.VMEM((DIM, LANES), jnp.float32),
        pltpu.VMEM((CH,), jnp.float32),
        pltpu.VMEM((CH,), jnp.float32),
        pltpu.SemaphoreType.DMA,
        pltpu.SemaphoreType.DMA,
        pltpu.SemaphoreType.DMA,
    ],
)
def _sc_call(ids_hbm, tok_hbm, comb_hbm, gam_hbm, bet_hbm, out_hbm,
             ids_v, rows0, rows1, ybuf, comb_v, gam_v, bet_v,
             mn_v, rs_v, gsem0, gsem1, osem):
    _sc_embed_ln(ids_hbm, tok_hbm, comb_hbm, gam_hbm, bet_hbm, out_hbm,
                 ids_v, rows0, rows1, ybuf, comb_v, gam_v, bet_v,
                 mn_v, rs_v, gsem0, gsem1, osem)


def kernel(input_ids, tok_table, pos_table, type_table, gamma, beta):
    b, seq = input_ids.shape
    ids2d = input_ids.reshape(-1).reshape(NTOK // 128, 128)
    combined = jnp.concatenate(
        [(pos_table[0] + type_table[0])[None, :],
         pos_table[1:1 + seq] + type_table[1][None, :]], axis=0)
    gam_b = jnp.broadcast_to(gamma[:, None], (DIM, LANES))
    bet_b = jnp.broadcast_to(beta[:, None], (DIM, LANES))
    out = _sc_call(ids2d, tok_table, combined, gam_b, bet_b)
    return out.reshape(b, seq, DIM)


# speedup vs baseline: 2.4950x; 2.3817x over previous
import functools

import jax
import jax.numpy as jnp
from jax import lax
from jax.experimental import pallas as pl
from jax.experimental.pallas import tpu as pltpu
from jax.experimental.pallas import tpu_sc as plsc

DIM = 64
LANES = 16
CH = 256
NW = 32
SEQ = 200
NTOK = 4096 * SEQ
TPW = NTOK // NW
NCH = TPW // CH
IDROWS = TPW // 128


def _sc_embed_ln(ids_hbm, tok_hbm, comb_hbm, gam_hbm, bet_hbm, out_hbm,
                 ids_v, rows0, rows1, ybuf, comb_v, gam_v, bet_v,
                 mn_v, rs_v, gsem0, gsem1, osem):
    wid = lax.axis_index("s") * 2 + lax.axis_index("c")
    base_tok = wid * TPW

    pltpu.sync_copy(ids_hbm.at[pl.ds(wid * IDROWS, IDROWS)], ids_v)
    pltpu.sync_copy(comb_hbm, comb_v)
    pltpu.sync_copy(gam_hbm, gam_v)
    pltpu.sync_copy(bet_hbm, bet_v)

    iota = lax.iota(jnp.int32, LANES)

    def fire_gather(c, rows, sem):
        for j in range(CH // 128):
            pltpu.async_copy(tok_hbm.at[ids_v.at[c * (CH // 128) + j]],
                             rows.at[pl.ds(j * 128, 128)], sem)

    def wait_gather(rows, sem):
        pltpu.make_async_copy(tok_hbm.at[pl.ds(0, CH)], rows, sem).wait()

    def fire_store(c, sem):
        pltpu.async_copy(ybuf, out_hbm.at[pl.ds(base_tok + c * CH, CH)], sem)

    def wait_store(sem):
        pltpu.make_async_copy(ybuf, out_hbm.at[pl.ds(0, CH)], sem).wait()

    half = jnp.float32(0.5)
    threehalf = jnp.float32(1.5)
    inv_dim = jnp.float32(1.0 / DIM)

    def pass1(c, rows):
        def group_body(g, _):
            row16 = g * LANES + iota
            loc = c * CH + row16
            ids16 = plsc.load_gather(ids_v, [loc // 128, loc % 128])
            flat = base_tok + loc
            l16 = flat % SEQ
            pos16 = jnp.where(ids16 == 0, 0, l16 + 1)

            def dbody(d, sqc):
                s, q, dcol = sqc
                tok = plsc.load_gather(rows, [row16, dcol])
                com = plsc.load_gather(comb_v, [pos16, dcol])
                v = tok + com
                plsc.store_scatter(rows, [row16, dcol], v)
                return s + v, q + v * v, (dcol + 1) & (DIM - 1)

            s, q, _ = lax.fori_loop(
                0, DIM, dbody,
                (jnp.zeros((LANES,), jnp.float32),
                 jnp.zeros((LANES,), jnp.float32),
                 iota),
                unroll=4)
            mean = s * inv_dim
            var = q * inv_dim - mean * mean
            x = var + jnp.float32(1e-12)
            i = lax.bitcast_convert_type(x, jnp.int32)
            i = jnp.int32(0x5F3759DF) - (i >> 1)
            y = lax.bitcast_convert_type(i, jnp.float32)
            y = y * (threehalf - half * x * y * y)
            y = y * (threehalf - half * x * y * y)
            y = y * (threehalf - half * x * y * y)
            mn_v[pl.ds(g * LANES, LANES)] = mean
            rs_v[pl.ds(g * LANES, LANES)] = y
            return 0

        lax.fori_loop(0, CH // LANES, group_body, 0)

    def pass2(rows):
        def group_body(g, _):
            row16 = g * LANES + iota
            mean = mn_v[pl.ds(g * LANES, LANES)]
            rstd = rs_v[pl.ds(g * LANES, LANES)]

            def dbody(d, dcol):
                a = rstd * plsc.load_gather(gam_v, [dcol])
                b = plsc.load_gather(bet_v, [dcol]) - mean * a
                v = plsc.load_gather(rows, [row16, dcol])
                plsc.store_scatter(ybuf, [row16, dcol], v * a + b)
                return (dcol + 1) & (DIM - 1)

            lax.fori_loop(0, DIM, dbody, iota, unroll=4)
            return 0

        lax.fori_loop(0, CH // LANES, group_body, 0)

    fire_gather(0, rows0, gsem0)

    def pair_body(p, _):
        c0 = p * 2
        c1 = c0 + 1
        wait_gather(rows0, gsem0)
        fire_gather(c1, rows1, gsem1)
        pass1(c0, rows0)

        @pl.when(p > 0)
        def _():
            wait_store(osem)

        pass2(rows0)
        fire_store(c0, osem)

        wait_gather(rows1, gsem1)

        @pl.when(p < NCH // 2 - 1)
        def _():
            fire_gather(c0 + 2, rows0, gsem0)

        pass1(c1, rows1)
        wait_store(osem)
        pass2(rows1)
        fire_store(c1, osem)
        return 0

    lax.fori_loop(0, NCH // 2, pair_body, 0)
    wait_store(osem)


@functools.partial(
    pl.kernel,
    mesh=plsc.VectorSubcoreMesh(core_axis_name="c", subcore_axis_name="s"),
    out_type=jax.ShapeDtypeStruct((NTOK, DIM), jnp.float32),
    compiler_params=pltpu.CompilerParams(use_tc_tiling_on_sc=False,
                                        needs_layout_passes=False),
    scratch_types=[
        pltpu.VMEM((IDROWS, 128), jnp.int32),
        pltpu.VMEM((CH, DIM), jnp.float32),
        pltpu.VMEM((CH, DIM), jnp.float32),
        pltpu.VMEM((CH, DIM), jnp.float32),
        pltpu.VMEM((SEQ + 1, DIM), jnp.float32),
        pltpu.VMEM((DIM,), jnp.float32),
        pltpu.VMEM((DIM,), jnp.float32),
        pltpu.VMEM((CH,), jnp.float32),
        pltpu.VMEM((CH,), jnp.float32),
        pltpu.SemaphoreType.DMA,
        pltpu.SemaphoreType.DMA,
        pltpu.SemaphoreType.DMA,
    ],
)
def _sc_call(ids_hbm, tok_hbm, comb_hbm, gam_hbm, bet_hbm, out_hbm,
             ids_v, rows0, rows1, ybuf, comb_v, gam_v, bet_v,
             mn_v, rs_v, gsem0, gsem1, osem):
    _sc_embed_ln(ids_hbm, tok_hbm, comb_hbm, gam_hbm, bet_hbm, out_hbm,
                 ids_v, rows0, rows1, ybuf, comb_v, gam_v, bet_v,
                 mn_v, rs_v, gsem0, gsem1, osem)


def kernel(input_ids, tok_table, pos_table, type_table, gamma, beta):
    b, seq = input_ids.shape
    ids2d = input_ids.reshape(-1).reshape(NTOK // 128, 128)
    combined = jnp.concatenate(
        [(pos_table[0] + type_table[0])[None, :],
         pos_table[1:1 + seq] + type_table[1][None, :]], axis=0)
    out = _sc_call(ids2d, tok_table, combined, gamma, beta)
    return out.reshape(b, seq, DIM)


# speedup vs baseline: 2.5616x; 1.0267x over previous
import functools

import jax
import jax.numpy as jnp
from jax import lax
from jax.experimental import pallas as pl
from jax.experimental.pallas import tpu as pltpu
from jax.experimental.pallas import tpu_sc as plsc

DIM = 64
LANES = 16
CH = 256
NW = 32
SEQ = 200
NTOK = 4096 * SEQ
TPW = NTOK // NW
NCH = TPW // CH
IDROWS = TPW // 128


def _sc_embed_ln(ids_hbm, tok_hbm, comb_hbm, gam_hbm, bet_hbm, out_hbm,
                 ids_v, rows0, rows1, ybuf, comb_v, gam_v, bet_v,
                 mn_v, rs_v, gsem0, gsem1, osem):
    wid = lax.axis_index("s") * 2 + lax.axis_index("c")
    base_tok = wid * TPW

    pltpu.sync_copy(ids_hbm.at[pl.ds(wid * IDROWS, IDROWS)], ids_v)
    pltpu.sync_copy(comb_hbm, comb_v)
    pltpu.sync_copy(gam_hbm, gam_v)
    pltpu.sync_copy(bet_hbm, bet_v)

    iota = lax.iota(jnp.int32, LANES)

    def fire_gather(c, rows, sem):
        for j in range(CH // 128):
            pltpu.async_copy(tok_hbm.at[ids_v.at[c * (CH // 128) + j]],
                             rows.at[pl.ds(j * 128, 128)], sem)

    def wait_gather(rows, sem):
        pltpu.make_async_copy(tok_hbm.at[pl.ds(0, CH)], rows, sem).wait()

    def fire_store(c, sem):
        pltpu.async_copy(ybuf, out_hbm.at[pl.ds(base_tok + c * CH, CH)], sem)

    def wait_store(sem):
        pltpu.make_async_copy(ybuf, out_hbm.at[pl.ds(0, CH)], sem).wait()

    half = jnp.float32(0.5)
    threehalf = jnp.float32(1.5)
    inv_dim = jnp.float32(1.0 / DIM)
    zf = jnp.zeros((LANES,), jnp.float32)

    def pass1(c, rows):
        def group_body(g, _):
            row16 = g * LANES + iota
            loc = c * CH + row16
            ids16 = plsc.load_gather(ids_v, [loc // 128, loc % 128])
            flat = base_tok + loc
            l16 = flat % SEQ
            pos16 = jnp.where(ids16 == 0, 0, l16 + 1)

            def dbody(i, carry):
                s0, q0, s1, q1, s2, q2, s3, q3, dcol = carry
                d1 = (dcol + 1) & (DIM - 1)
                d2 = (dcol + 2) & (DIM - 1)
                d3 = (dcol + 3) & (DIM - 1)
                v0 = (plsc.load_gather(rows, [row16, dcol])
                      + plsc.load_gather(comb_v, [pos16, dcol]))
                plsc.store_scatter(rows, [row16, dcol], v0)
                v1 = (plsc.load_gather(rows, [row16, d1])
                      + plsc.load_gather(comb_v, [pos16, d1]))
                plsc.store_scatter(rows, [row16, d1], v1)
                v2 = (plsc.load_gather(rows, [row16, d2])
                      + plsc.load_gather(comb_v, [pos16, d2]))
                plsc.store_scatter(rows, [row16, d2], v2)
                v3 = (plsc.load_gather(rows, [row16, d3])
                      + plsc.load_gather(comb_v, [pos16, d3]))
                plsc.store_scatter(rows, [row16, d3], v3)
                return (s0 + v0, q0 + v0 * v0, s1 + v1, q1 + v1 * v1,
                        s2 + v2, q2 + v2 * v2, s3 + v3, q3 + v3 * v3,
                        (dcol + 4) & (DIM - 1))

            s0, q0, s1, q1, s2, q2, s3, q3, _ = lax.fori_loop(
                0, DIM // 4, dbody,
                (zf, zf, zf, zf, zf, zf, zf, zf, iota))
            s = (s0 + s1) + (s2 + s3)
            q = (q0 + q1) + (q2 + q3)
            mean = s * inv_dim
            var = q * inv_dim - mean * mean
            x = var + jnp.float32(1e-12)
            i = lax.bitcast_convert_type(x, jnp.int32)
            i = jnp.int32(0x5F3759DF) - (i >> 1)
            y = lax.bitcast_convert_type(i, jnp.float32)
            y = y * (threehalf - half * x * y * y)
            y = y * (threehalf - half * x * y * y)
            y = y * (threehalf - half * x * y * y)
            mn_v[pl.ds(g * LANES, LANES)] = mean
            rs_v[pl.ds(g * LANES, LANES)] = y
            return 0

        lax.fori_loop(0, CH // LANES, group_body, 0)

    def pass2(rows):
        gk = [gam_v[pl.ds(k * LANES, LANES)] for k in range(DIM // LANES)]
        bk = [bet_v[pl.ds(k * LANES, LANES)] for k in range(DIM // LANES)]

        def group_body(g, _):
            mean = mn_v[pl.ds(g * LANES, LANES)]
            rstd = rs_v[pl.ds(g * LANES, LANES)]
            t0 = g * LANES
            for l in range(LANES):
                bm = jnp.broadcast_to(mean[l], (LANES,))
                br = jnp.broadcast_to(rstd[l], (LANES,))
                for k in range(DIM // LANES):
                    x = rows[t0 + l, pl.ds(k * LANES, LANES)]
                    ybuf[t0 + l, pl.ds(k * LANES, LANES)] = (
                        ((x - bm) * br) * gk[k] + bk[k])
            return 0

        lax.fori_loop(0, CH // LANES, group_body, 0)

    fire_gather(0, rows0, gsem0)

    def pair_body(p, _):
        c0 = p * 2
        c1 = c0 + 1
        wait_gather(rows0, gsem0)
        fire_gather(c1, rows1, gsem1)
        pass1(c0, rows0)

        @pl.when(p > 0)
        def _():
            wait_store(osem)

        pass2(rows0)
        fire_store(c0, osem)

        wait_gather(rows1, gsem1)

        @pl.when(p < NCH // 2 - 1)
        def _():
            fire_gather(c0 + 2, rows0, gsem0)

        pass1(c1, rows1)
        wait_store(osem)
        pass2(rows1)
        fire_store(c1, osem)
        return 0

    lax.fori_loop(0, NCH // 2, pair_body, 0)
    wait_store(osem)


@functools.partial(
    pl.kernel,
    mesh=plsc.VectorSubcoreMesh(core_axis_name="c", subcore_axis_name="s"),
    out_type=jax.ShapeDtypeStruct((NTOK, DIM), jnp.float32),
    compiler_params=pltpu.CompilerParams(use_tc_tiling_on_sc=False,
                                        needs_layout_passes=False),
    scratch_types=[
        pltpu.VMEM((IDROWS, 128), jnp.int32),
        pltpu.VMEM((CH, DIM), jnp.float32),
        pltpu.VMEM((CH, DIM), jnp.float32),
        pltpu.VMEM((CH, DIM), jnp.float32),
        pltpu.VMEM((SEQ + 1, DIM), jnp.float32),
        pltpu.VMEM((DIM,), jnp.float32),
        pltpu.VMEM((DIM,), jnp.float32),
        pltpu.VMEM((CH,), jnp.float32),
        pltpu.VMEM((CH,), jnp.float32),
        pltpu.SemaphoreType.DMA,
        pltpu.SemaphoreType.DMA,
        pltpu.SemaphoreType.DMA,
    ],
)
def _sc_call(ids_hbm, tok_hbm, comb_hbm, gam_hbm, bet_hbm, out_hbm,
             ids_v, rows0, rows1, ybuf, comb_v, gam_v, bet_v,
             mn_v, rs_v, gsem0, gsem1, osem):
    _sc_embed_ln(ids_hbm, tok_hbm, comb_hbm, gam_hbm, bet_hbm, out_hbm,
                 ids_v, rows0, rows1, ybuf, comb_v, gam_v, bet_v,
                 mn_v, rs_v, gsem0, gsem1, osem)


def kernel(input_ids, tok_table, pos_table, type_table, gamma, beta):
    b, seq = input_ids.shape
    ids2d = input_ids.reshape(-1).reshape(NTOK // 128, 128)
    combined = jnp.concatenate(
        [(pos_table[0] + type_table[0])[None, :],
         pos_table[1:1 + seq] + type_table[1][None, :]], axis=0)
    out = _sc_call(ids2d, tok_table, combined, gamma, beta)
    return out.reshape(b, seq, DIM)


# speedup vs baseline: 4.0464x; 1.5796x over previous
import functools

import jax
import jax.numpy as jnp
from jax import lax
from jax.experimental import pallas as pl
from jax.experimental.pallas import tpu as pltpu
from jax.experimental.pallas import tpu_sc as plsc

DIM = 64
LANES = 16
CH = 256
NW = 32
SEQ = 200
NTOK = 4096 * SEQ
TPW = NTOK // NW
NCH = TPW // CH
IDROWS = TPW // 128


def _sc_embed_ln(ids_hbm, tok_hbm, comb_hbm, gam_hbm, bet_hbm, out_hbm,
                 ids_v, rows0, rows1, ybuf, comb_v, gam_v, bet_v,
                 mn_v, rs_v, gsem0, gsem1, osem):
    wid = lax.axis_index("s") * 2 + lax.axis_index("c")
    base_tok = wid * TPW

    pltpu.sync_copy(ids_hbm.at[pl.ds(wid * IDROWS, IDROWS)], ids_v)
    pltpu.sync_copy(comb_hbm, comb_v)
    pltpu.sync_copy(gam_hbm, gam_v)
    pltpu.sync_copy(bet_hbm, bet_v)

    iota = lax.iota(jnp.int32, LANES)

    def fire_gather(c, rows, sem):
        for j in range(CH // 128):
            pltpu.async_copy(tok_hbm.at[ids_v.at[c * (CH // 128) + j]],
                             rows.at[pl.ds(j * 128, 128)], sem)

    def wait_gather(rows, sem):
        pltpu.make_async_copy(tok_hbm.at[pl.ds(0, CH)], rows, sem).wait()

    def fire_store(c, sem):
        pltpu.async_copy(ybuf, out_hbm.at[pl.ds(base_tok + c * CH, CH)], sem)

    def wait_store(sem):
        pltpu.make_async_copy(ybuf, out_hbm.at[pl.ds(0, CH)], sem).wait()

    def lane_gather(vec, lidx):
        return lax.gather(
            vec, lidx[:, None],
            lax.GatherDimensionNumbers(offset_dims=(),
                                       collapsed_slice_dims=(0,),
                                       start_index_map=(0,)),
            (1,), mode=lax.GatherScatterMode.PROMISE_IN_BOUNDS)

    half = jnp.float32(0.5)
    threehalf = jnp.float32(1.5)
    inv_dim = jnp.float32(1.0 / DIM)
    zf = jnp.zeros((LANES,), jnp.float32)

    def pass1(c, rows):
        def group_body(g, _):
            row16 = g * LANES + iota
            loc = c * CH + row16
            ids16 = plsc.load_gather(ids_v, [loc // 128, loc % 128])
            flat = base_tok + loc
            l16 = flat % SEQ
            pos16 = jnp.where(ids16 == 0, 0, l16 + 1)

            U = 8

            def dbody(i, carry):
                ss = carry[:U]
                qs = carry[U:2 * U]
                dcol = carry[2 * U]
                cols = [dcol] + [(dcol + j) & (DIM - 1) for j in range(1, U)]
                toks = [plsc.load_gather(rows, [row16, c]) for c in cols]
                combs = [plsc.load_gather(comb_v, [pos16, c]) for c in cols]
                vs = [t + cm for t, cm in zip(toks, combs)]
                for c, v in zip(cols, vs):
                    plsc.store_scatter(rows, [row16, c], v)
                ss = tuple(s + v for s, v in zip(ss, vs))
                qs = tuple(q + v * v for q, v in zip(qs, vs))
                return ss + qs + ((dcol + U) & (DIM - 1),)

            carry = lax.fori_loop(0, DIM // U, dbody,
                                  (zf,) * (2 * U) + (iota,))
            ss = carry[:U]
            qs = carry[U:2 * U]
            s = ((ss[0] + ss[1]) + (ss[2] + ss[3])) + \
                ((ss[4] + ss[5]) + (ss[6] + ss[7]))
            q = ((qs[0] + qs[1]) + (qs[2] + qs[3])) + \
                ((qs[4] + qs[5]) + (qs[6] + qs[7]))
            mean = s * inv_dim
            var = q * inv_dim - mean * mean
            x = var + jnp.float32(1e-12)
            i = lax.bitcast_convert_type(x, jnp.int32)
            i = jnp.int32(0x5F3759DF) - (i >> 1)
            y = lax.bitcast_convert_type(i, jnp.float32)
            y = y * (threehalf - half * x * y * y)
            y = y * (threehalf - half * x * y * y)
            y = y * (threehalf - half * x * y * y)
            mn_v[pl.ds(g * LANES, LANES)] = mean
            rs_v[pl.ds(g * LANES, LANES)] = y
            return 0

        lax.fori_loop(0, CH // LANES, group_body, 0)

    def pass2(rows):
        gk = [gam_v[pl.ds(k * LANES, LANES)] for k in range(DIM // LANES)]
        bk = [bet_v[pl.ds(k * LANES, LANES)] for k in range(DIM // LANES)]

        def group_body(g, _):
            mean = mn_v[pl.ds(g * LANES, LANES)]
            rstd = rs_v[pl.ds(g * LANES, LANES)]
            t0 = g * LANES
            TB = 4
            for l in range(0, LANES, TB):
                bms = [lane_gather(mean, jnp.full((LANES,), l + j,
                                                  dtype=jnp.int32))
                       for j in range(TB)]
                brs = [lane_gather(rstd, jnp.full((LANES,), l + j,
                                                  dtype=jnp.int32))
                       for j in range(TB)]
                for k in range(DIM // LANES):
                    xs = [rows[t0 + l + j, pl.ds(k * LANES, LANES)]
                          for j in range(TB)]
                    ys = [((x - bm) * br) * gk[k] + bk[k]
                          for x, bm, br in zip(xs, bms, brs)]
                    for j in range(TB):
                        ybuf[t0 + l + j, pl.ds(k * LANES, LANES)] = ys[j]
            return 0

        lax.fori_loop(0, CH // LANES, group_body, 0)

    fire_gather(0, rows0, gsem0)

    def pair_body(p, _):
        c0 = p * 2
        c1 = c0 + 1
        wait_gather(rows0, gsem0)
        fire_gather(c1, rows1, gsem1)
        pass1(c0, rows0)

        @pl.when(p > 0)
        def _():
            wait_store(osem)

        pass2(rows0)
        fire_store(c0, osem)

        wait_gather(rows1, gsem1)

        @pl.when(p < NCH // 2 - 1)
        def _():
            fire_gather(c0 + 2, rows0, gsem0)

        pass1(c1, rows1)
        wait_store(osem)
        pass2(rows1)
        fire_store(c1, osem)
        return 0

    lax.fori_loop(0, NCH // 2, pair_body, 0)
    wait_store(osem)


@functools.partial(
    pl.kernel,
    mesh=plsc.VectorSubcoreMesh(core_axis_name="c", subcore_axis_name="s"),
    out_type=jax.ShapeDtypeStruct((NTOK, DIM), jnp.float32),
    compiler_params=pltpu.CompilerParams(use_tc_tiling_on_sc=False,
                                        needs_layout_passes=False),
    scratch_types=[
        pltpu.VMEM((IDROWS, 128), jnp.int32),
        pltpu.VMEM((CH, DIM), jnp.float32),
        pltpu.VMEM((CH, DIM), jnp.float32),
        pltpu.VMEM((CH, DIM), jnp.float32),
        pltpu.VMEM((SEQ + 1, DIM), jnp.float32),
        pltpu.VMEM((DIM,), jnp.float32),
        pltpu.VMEM((DIM,), jnp.float32),
        pltpu.VMEM((CH,), jnp.float32),
        pltpu.VMEM((CH,), jnp.float32),
        pltpu.SemaphoreType.DMA,
        pltpu.SemaphoreType.DMA,
        pltpu.SemaphoreType.DMA,
    ],
)
def _sc_call(ids_hbm, tok_hbm, comb_hbm, gam_hbm, bet_hbm, out_hbm,
             ids_v, rows0, rows1, ybuf, comb_v, gam_v, bet_v,
             mn_v, rs_v, gsem0, gsem1, osem):
    _sc_embed_ln(ids_hbm, tok_hbm, comb_hbm, gam_hbm, bet_hbm, out_hbm,
                 ids_v, rows0, rows1, ybuf, comb_v, gam_v, bet_v,
                 mn_v, rs_v, gsem0, gsem1, osem)


def kernel(input_ids, tok_table, pos_table, type_table, gamma, beta):
    b, seq = input_ids.shape
    ids2d = input_ids.reshape(-1).reshape(NTOK // 128, 128)
    combined = jnp.concatenate(
        [(pos_table[0] + type_table[0])[None, :],
         pos_table[1:1 + seq] + type_table[1][None, :]], axis=0)
    out = _sc_call(ids2d, tok_table, combined, gamma, beta)
    return out.reshape(b, seq, DIM)


# speedup vs baseline: 4.9997x; 1.2356x over previous
import functools

import jax
import jax.numpy as jnp
from jax import lax
from jax.experimental import pallas as pl
from jax.experimental.pallas import tpu as pltpu
from jax.experimental.pallas import tpu_sc as plsc

DIM = 64
LANES = 16
CH = 256
NW = 32
SEQ = 200
NTOK = 4096 * SEQ
TPW = NTOK // NW
NCH = TPW // CH
IDROWS = TPW // 128


def _sc_embed_ln(ids_hbm, tok_hbm, comb_hbm, gam_hbm, bet_hbm, out_hbm,
                 ids_v, rows0, rows1, ybuf,
                 comb_v, gam_v, bet_v, mn_v, rs_v, gsem0, gsem1, osem):
    wid = lax.axis_index("s") * 2 + lax.axis_index("c")
    base_tok = wid * TPW

    pltpu.sync_copy(ids_hbm.at[pl.ds(wid * IDROWS, IDROWS)], ids_v)
    pltpu.sync_copy(comb_hbm, comb_v)
    pltpu.sync_copy(gam_hbm, gam_v)
    pltpu.sync_copy(bet_hbm, bet_v)

    iota = lax.iota(jnp.int32, LANES)

    def fire_gather(c, rows, sem):
        for j in range(CH // 128):
            pltpu.async_copy(tok_hbm.at[ids_v.at[c * (CH // 128) + j]],
                             rows.at[pl.ds(j * 128, 128)], sem)

    def wait_gather(rows, sem):
        pltpu.make_async_copy(tok_hbm.at[pl.ds(0, CH)], rows, sem).wait()

    def fire_store(c, sem):
        pltpu.async_copy(
            ybuf, out_hbm.at[pl.ds(base_tok + c * CH, CH), pl.ds(0, DIM)],
            sem)

    def wait_store(sem):
        pltpu.make_async_copy(
            ybuf, out_hbm.at[pl.ds(0, CH), pl.ds(0, DIM)], sem).wait()

    def lane_gather(vec, lidx):
        return lax.gather(
            vec, lidx[:, None],
            lax.GatherDimensionNumbers(offset_dims=(),
                                       collapsed_slice_dims=(0,),
                                       start_index_map=(0,)),
            (1,), mode=lax.GatherScatterMode.PROMISE_IN_BOUNDS)

    half = jnp.float32(0.5)
    threehalf = jnp.float32(1.5)
    inv_dim = jnp.float32(1.0 / DIM)
    zf = jnp.zeros((LANES,), jnp.float32)

    def pass1(c, rows):
        def group_body(g, _):
            row16 = g * LANES + iota
            loc = c * CH + row16
            ids16 = plsc.load_gather(ids_v, [loc // 128, loc % 128])
            flat = base_tok + loc
            l16 = flat % SEQ
            pos16 = jnp.where(ids16 == 0, 0, l16 + 1)

            U = 8

            def dbody(i, carry):
                ss = carry[:U]
                qs = carry[U:2 * U]
                dcol = carry[2 * U]
                cols = [dcol] + [(dcol + j) & (DIM - 1) for j in range(1, U)]
                toks = [plsc.load_gather(rows, [row16, col])
                        for col in cols]
                combs = [plsc.load_gather(comb_v, [pos16, col])
                         for col in cols]
                vs = [t + cm for t, cm in zip(toks, combs)]
                for col, v in zip(cols, vs):
                    plsc.store_scatter(rows, [row16, col], v)
                ss = tuple(s + v for s, v in zip(ss, vs))
                qs = tuple(q + v * v for q, v in zip(qs, vs))
                return ss + qs + ((dcol + U) & (DIM - 1),)

            carry = lax.fori_loop(0, DIM // U, dbody,
                                  (zf,) * (2 * U) + (iota,))
            ss = carry[:U]
            qs = carry[U:2 * U]
            s = ((ss[0] + ss[1]) + (ss[2] + ss[3])) + \
                ((ss[4] + ss[5]) + (ss[6] + ss[7]))
            q = ((qs[0] + qs[1]) + (qs[2] + qs[3])) + \
                ((qs[4] + qs[5]) + (qs[6] + qs[7]))
            mean = s * inv_dim
            var = q * inv_dim - mean * mean
            x = var + jnp.float32(1e-12)
            i = lax.bitcast_convert_type(x, jnp.int32)
            i = jnp.int32(0x5F3759DF) - (i >> 1)
            y = lax.bitcast_convert_type(i, jnp.float32)
            y = y * (threehalf - half * x * y * y)
            y = y * (threehalf - half * x * y * y)
            y = y * (threehalf - half * x * y * y)
            mn_v[pl.ds(g * LANES, LANES)] = mean
            rs_v[pl.ds(g * LANES, LANES)] = y
            return 0

        lax.fori_loop(0, CH // LANES, group_body, 0)

    def pass2(rows):
        gk = [gam_v[pl.ds(k * LANES, LANES)] for k in range(DIM // LANES)]
        bk = [bet_v[pl.ds(k * LANES, LANES)] for k in range(DIM // LANES)]

        def group_body(g, _):
            mean = mn_v[pl.ds(g * LANES, LANES)]
            rstd = rs_v[pl.ds(g * LANES, LANES)]
            t0 = g * LANES
            TB = 4
            for l in range(0, LANES, TB):
                bms = [lane_gather(mean, jnp.full((LANES,), l + j,
                                                  dtype=jnp.int32))
                       for j in range(TB)]
                brs = [lane_gather(rstd, jnp.full((LANES,), l + j,
                                                  dtype=jnp.int32))
                       for j in range(TB)]
                for k in range(DIM // LANES):
                    xs = [rows[t0 + l + j, pl.ds(k * LANES, LANES)]
                          for j in range(TB)]
                    ys = [((x - bm) * br) * gk[k] + bk[k]
                          for x, bm, br in zip(xs, bms, brs)]
                    for j in range(TB):
                        ybuf[t0 + l + j, pl.ds(k * LANES, LANES)] = ys[j]
            return 0

        lax.fori_loop(0, CH // LANES, group_body, 0)

    fire_gather(0, rows0, gsem0)

    def pair_body(p, _):
        c0 = p * 2
        c1 = c0 + 1
        wait_gather(rows0, gsem0)
        fire_gather(c1, rows1, gsem1)
        pass1(c0, rows0)

        @pl.when(p > 0)
        def _():
            wait_store(osem)

        pass2(rows0)
        fire_store(c0, osem)

        wait_gather(rows1, gsem1)

        @pl.when(p < NCH // 2 - 1)
        def _():
            fire_gather(c0 + 2, rows0, gsem0)

        pass1(c1, rows1)
        wait_store(osem)
        pass2(rows1)
        fire_store(c1, osem)
        return 0

    lax.fori_loop(0, NCH // 2, pair_body, 0)
    wait_store(osem)


@functools.partial(
    pl.kernel,
    mesh=plsc.VectorSubcoreMesh(core_axis_name="c", subcore_axis_name="s"),
    out_type=jax.ShapeDtypeStruct((NTOK, 128), jnp.float32),
    compiler_params=pltpu.CompilerParams(use_tc_tiling_on_sc=False,
                                        needs_layout_passes=False),
    scratch_types=[
        pltpu.VMEM((IDROWS, 128), jnp.int32),
        pltpu.VMEM((CH, DIM), jnp.float32),
        pltpu.VMEM((CH, DIM), jnp.float32),
        pltpu.VMEM((CH, DIM), jnp.float32),
        pltpu.VMEM((SEQ + 1, DIM), jnp.float32),
        pltpu.VMEM((DIM,), jnp.float32),
        pltpu.VMEM((DIM,), jnp.float32),
        pltpu.VMEM((CH,), jnp.float32),
        pltpu.VMEM((CH,), jnp.float32),
        pltpu.SemaphoreType.DMA,
        pltpu.SemaphoreType.DMA,
        pltpu.SemaphoreType.DMA,
    ],
)
def _sc_call(ids_hbm, tok_hbm, comb_hbm, gam_hbm, bet_hbm, out_hbm,
             ids_v, rows0, rows1, ybuf,
             comb_v, gam_v, bet_v, mn_v, rs_v, gsem0, gsem1, osem):
    _sc_embed_ln(ids_hbm, tok_hbm, comb_hbm, gam_hbm, bet_hbm, out_hbm,
                 ids_v, rows0, rows1, ybuf,
                 comb_v, gam_v, bet_v, mn_v, rs_v, gsem0, gsem1, osem)


def kernel(input_ids, tok_table, pos_table, type_table, gamma, beta):
    b, seq = input_ids.shape
    ids2d = input_ids.reshape(-1).reshape(NTOK // 128, 128)
    combined = jnp.concatenate(
        [(pos_table[0] + type_table[0])[None, :],
         pos_table[1:1 + seq] + type_table[1][None, :]], axis=0)
    out = _sc_call(ids2d, tok_table, combined, gamma, beta)
    return out[:, :DIM].reshape(b, seq, DIM)


# speedup vs baseline: 5.2454x; 1.0491x over previous
import functools

import jax
import jax.numpy as jnp
from jax import lax
from jax.experimental import pallas as pl
from jax.experimental.pallas import tpu as pltpu
from jax.experimental.pallas import tpu_sc as plsc

DIM = 64
LANES = 16
CH = 256
NW = 32
SEQ = 200
NTOK = 4096 * SEQ
TPW = NTOK // NW
NCH = TPW // CH
IDROWS = TPW // 128


def _sc_embed_ln(ids_hbm, tok_hbm, comb_hbm, gam_hbm, bet_hbm, out_hbm,
                 ids_v, rows0, rows1, ybuf,
                 comb_v, gam_v, bet_v, mn_v, rs_v, gsem0, gsem1, osem):
    wid = lax.axis_index("s") * 2 + lax.axis_index("c")
    base_tok = wid * TPW

    pltpu.sync_copy(ids_hbm.at[pl.ds(wid * IDROWS, IDROWS)], ids_v)
    pltpu.sync_copy(comb_hbm, comb_v)
    pltpu.sync_copy(gam_hbm, gam_v)
    pltpu.sync_copy(bet_hbm, bet_v)

    iota = lax.iota(jnp.int32, LANES)

    def fire_gather(c, rows, sem):
        for j in range(CH // 128):
            pltpu.async_copy(tok_hbm.at[ids_v.at[c * (CH // 128) + j]],
                             rows.at[pl.ds(j * 128, 128)], sem)

    def wait_gather(rows, sem):
        pltpu.make_async_copy(tok_hbm.at[pl.ds(0, CH)], rows, sem).wait()

    def fire_store(c, sem):
        pltpu.async_copy(
            ybuf, out_hbm.at[pl.ds(base_tok + c * CH, CH), pl.ds(0, DIM)],
            sem)

    def wait_store(sem):
        pltpu.make_async_copy(
            ybuf, out_hbm.at[pl.ds(0, CH), pl.ds(0, DIM)], sem).wait()

    def lane_gather(vec, lidx):
        return lax.gather(
            vec, lidx[:, None],
            lax.GatherDimensionNumbers(offset_dims=(),
                                       collapsed_slice_dims=(0,),
                                       start_index_map=(0,)),
            (1,), mode=lax.GatherScatterMode.PROMISE_IN_BOUNDS)

    half = jnp.float32(0.5)
    threehalf = jnp.float32(1.5)
    inv_dim = jnp.float32(1.0 / DIM)
    zf = jnp.zeros((LANES,), jnp.float32)

    def pass1(c, rows):
        def group_body(g, _):
            row16 = g * LANES + iota
            loc = c * CH + row16
            ids16 = plsc.load_gather(ids_v, [loc // 128, loc % 128])
            flat = base_tok + loc
            l16 = flat % SEQ
            pos16 = jnp.where(ids16 == 0, 0, l16 + 1)

            U = 8

            def dbody(i, carry):
                ss = carry[:U]
                qs = carry[U:2 * U]
                dcol = carry[2 * U]
                cols = [dcol] + [(dcol + j) & (DIM - 1) for j in range(1, U)]
                toks = [plsc.load_gather(rows, [row16, col])
                        for col in cols]
                combs = [plsc.load_gather(comb_v, [pos16, col])
                         for col in cols]
                vs = [t + cm for t, cm in zip(toks, combs)]
                for col, v in zip(cols, vs):
                    plsc.store_scatter(rows, [row16, col], v)
                ss = tuple(s + v for s, v in zip(ss, vs))
                qs = tuple(q + v * v for q, v in zip(qs, vs))
                return ss + qs + ((dcol + U) & (DIM - 1),)

            carry = lax.fori_loop(0, DIM // U, dbody,
                                  (zf,) * (2 * U) + (iota,))
            ss = carry[:U]
            qs = carry[U:2 * U]
            s = ((ss[0] + ss[1]) + (ss[2] + ss[3])) + \
                ((ss[4] + ss[5]) + (ss[6] + ss[7]))
            q = ((qs[0] + qs[1]) + (qs[2] + qs[3])) + \
                ((qs[4] + qs[5]) + (qs[6] + qs[7]))
            mean = s * inv_dim
            var = q * inv_dim - mean * mean
            x = var + jnp.float32(1e-12)
            i = lax.bitcast_convert_type(x, jnp.int32)
            i = jnp.int32(0x5F3759DF) - (i >> 1)
            y = lax.bitcast_convert_type(i, jnp.float32)
            y = y * (threehalf - half * x * y * y)
            y = y * (threehalf - half * x * y * y)
            y = y * (threehalf - half * x * y * y)
            mn_v[pl.ds(g * LANES, LANES)] = mean
            rs_v[pl.ds(g * LANES, LANES)] = y
            return 0

        lax.fori_loop(0, CH // LANES, group_body, 0)

    def pass2(rows):
        gk = [gam_v[pl.ds(k * LANES, LANES)] for k in range(DIM // LANES)]
        bk = [bet_v[pl.ds(k * LANES, LANES)] for k in range(DIM // LANES)]

        def group_body(g, _):
            mean = mn_v[pl.ds(g * LANES, LANES)]
            rstd = rs_v[pl.ds(g * LANES, LANES)]
            t0 = g * LANES
            TB = 4
            for l in range(0, LANES, TB):
                bms = [lane_gather(mean, jnp.full((LANES,), l + j,
                                                  dtype=jnp.int32))
                       for j in range(TB)]
                brs = [lane_gather(rstd, jnp.full((LANES,), l + j,
                                                  dtype=jnp.int32))
                       for j in range(TB)]
                for k in range(DIM // LANES):
                    xs = [rows[t0 + l + j, pl.ds(k * LANES, LANES)]
                          for j in range(TB)]
                    ys = [((x - bm) * br) * gk[k] + bk[k]
                          for x, bm, br in zip(xs, bms, brs)]
                    for j in range(TB):
                        ybuf[t0 + l + j, pl.ds(k * LANES, LANES)] = ys[j]
            return 0

        lax.fori_loop(0, CH // LANES, group_body, 0)

    fire_gather(0, rows0, gsem0)

    def pair_body(p, _):
        c0 = p * 2
        c1 = c0 + 1
        wait_gather(rows0, gsem0)
        fire_gather(c1, rows1, gsem1)
        pass1(c0, rows0)

        @pl.when(p > 0)
        def _():
            wait_store(osem)

        pass2(rows0)
        fire_store(c0, osem)

        wait_gather(rows1, gsem1)

        @pl.when(p < NCH // 2 - 1)
        def _():
            fire_gather(c0 + 2, rows0, gsem0)

        pass1(c1, rows1)
        wait_store(osem)
        pass2(rows1)
        fire_store(c1, osem)
        return 0

    lax.fori_loop(0, NCH // 2, pair_body, 0)
    wait_store(osem)


@functools.partial(
    pl.kernel,
    mesh=plsc.VectorSubcoreMesh(core_axis_name="c", subcore_axis_name="s"),
    out_type=jax.ShapeDtypeStruct((NTOK, 128), jnp.float32),
    compiler_params=pltpu.CompilerParams(use_tc_tiling_on_sc=False,
                                        needs_layout_passes=False),
    scratch_types=[
        pltpu.VMEM((IDROWS, 128), jnp.int32),
        pltpu.VMEM((CH, 128), jnp.float32),
        pltpu.VMEM((CH, 128), jnp.float32),
        pltpu.VMEM((CH, DIM), jnp.float32),
        pltpu.VMEM((SEQ + 1, DIM), jnp.float32),
        pltpu.VMEM((DIM,), jnp.float32),
        pltpu.VMEM((DIM,), jnp.float32),
        pltpu.VMEM((CH,), jnp.float32),
        pltpu.VMEM((CH,), jnp.float32),
        pltpu.SemaphoreType.DMA,
        pltpu.SemaphoreType.DMA,
        pltpu.SemaphoreType.DMA,
    ],
)
def _sc_call(ids_hbm, tok_hbm, comb_hbm, gam_hbm, bet_hbm, out_hbm,
             ids_v, rows0, rows1, ybuf,
             comb_v, gam_v, bet_v, mn_v, rs_v, gsem0, gsem1, osem):
    _sc_embed_ln(ids_hbm, tok_hbm, comb_hbm, gam_hbm, bet_hbm, out_hbm,
                 ids_v, rows0, rows1, ybuf,
                 comb_v, gam_v, bet_v, mn_v, rs_v, gsem0, gsem1, osem)


def kernel(input_ids, tok_table, pos_table, type_table, gamma, beta):
    b, seq = input_ids.shape
    ids2d = input_ids.reshape(-1).reshape(NTOK // 128, 128)
    combined = jnp.concatenate(
        [(pos_table[0] + type_table[0])[None, :],
         pos_table[1:1 + seq] + type_table[1][None, :]], axis=0)
    tokp = jnp.pad(tok_table, ((0, 0), (0, 64)))
    out = _sc_call(ids2d, tokp, combined, gamma, beta)
    return out[:, :DIM].reshape(b, seq, DIM)


# speedup vs baseline: 5.6476x; 1.0767x over previous
import functools

import jax
import jax.numpy as jnp
from jax import lax
from jax.experimental import pallas as pl
from jax.experimental.pallas import tpu as pltpu
from jax.experimental.pallas import tpu_sc as plsc

DIM = 64
LANES = 16
CH = 256
NW = 32
SEQ = 200
NTOK = 4096 * SEQ
TPW = NTOK // NW
NCH = TPW // CH
IDROWS = TPW // 128


def _sc_embed_ln(ids_hbm, tok_hbm, comb_hbm, gam_hbm, bet_hbm, out_hbm,
                 ids_v, rows0, rows1, ybuf,
                 comb_v, gam_v, bet_v, mn_v, rs_v, gsem0, gsem1, osem):
    wid = lax.axis_index("s") * 2 + lax.axis_index("c")
    base_tok = wid * TPW

    pltpu.sync_copy(ids_hbm.at[pl.ds(wid * IDROWS, IDROWS)], ids_v)
    pltpu.sync_copy(comb_hbm, comb_v)
    pltpu.sync_copy(gam_hbm, gam_v)
    pltpu.sync_copy(bet_hbm, bet_v)

    iota = lax.iota(jnp.int32, LANES)

    def fire_gather(c, rows, sem):
        for j in range(CH // 128):
            pltpu.async_copy(tok_hbm.at[ids_v.at[c * (CH // 128) + j]],
                             rows.at[pl.ds(j * 128, 128)], sem)

    def wait_gather(rows, sem):
        pltpu.make_async_copy(tok_hbm.at[pl.ds(0, CH)], rows, sem).wait()

    def fire_store(c, sem):
        pltpu.async_copy(
            ybuf, out_hbm.at[pl.ds(base_tok + c * CH, CH), pl.ds(0, DIM)],
            sem)

    def wait_store(sem):
        pltpu.make_async_copy(
            ybuf, out_hbm.at[pl.ds(0, CH), pl.ds(0, DIM)], sem).wait()

    def lane_gather(vec, lidx):
        return lax.gather(
            vec, lidx[:, None],
            lax.GatherDimensionNumbers(offset_dims=(),
                                       collapsed_slice_dims=(0,),
                                       start_index_map=(0,)),
            (1,), mode=lax.GatherScatterMode.PROMISE_IN_BOUNDS)

    half = jnp.float32(0.5)
    threehalf = jnp.float32(1.5)
    inv_dim = jnp.float32(1.0 / DIM)
    zf = jnp.zeros((LANES,), jnp.float32)

    def pass1(c, rows):
        def group_body(g, _):
            row16 = g * LANES + iota
            loc = c * CH + row16
            ids16 = plsc.load_gather(ids_v, [loc // 128, loc % 128])
            flat = base_tok + loc
            l16 = flat % SEQ
            pos16 = jnp.where(ids16 == 0, 0, l16 + 1)

            U = 8

            def dbody(i, carry):
                ss = carry[:U]
                qs = carry[U:2 * U]
                dcol = carry[2 * U]
                cols = [dcol] + [(dcol + j) & (DIM - 1) for j in range(1, U)]
                toks = [plsc.load_gather(rows, [row16, col])
                        for col in cols]
                combs = [plsc.load_gather(comb_v, [pos16, col])
                         for col in cols]
                vs = [t + cm for t, cm in zip(toks, combs)]
                for col, v in zip(cols, vs):
                    plsc.store_scatter(rows, [row16, col], v)
                ss = tuple(s + v for s, v in zip(ss, vs))
                qs = tuple(q + v * v for q, v in zip(qs, vs))
                return ss + qs + ((dcol + U) & (DIM - 1),)

            carry = lax.fori_loop(0, DIM // U, dbody,
                                  (zf,) * (2 * U) + (iota,))
            ss = carry[:U]
            qs = carry[U:2 * U]
            s = ((ss[0] + ss[1]) + (ss[2] + ss[3])) + \
                ((ss[4] + ss[5]) + (ss[6] + ss[7]))
            q = ((qs[0] + qs[1]) + (qs[2] + qs[3])) + \
                ((qs[4] + qs[5]) + (qs[6] + qs[7]))
            mean = s * inv_dim
            var = q * inv_dim - mean * mean
            x = var + jnp.float32(1e-12)
            i = lax.bitcast_convert_type(x, jnp.int32)
            i = jnp.int32(0x5F3759DF) - (i >> 1)
            y = lax.bitcast_convert_type(i, jnp.float32)
            y = y * (threehalf - half * x * y * y)
            y = y * (threehalf - half * x * y * y)
            mn_v[pl.ds(g * LANES, LANES)] = mean
            rs_v[pl.ds(g * LANES, LANES)] = y
            return 0

        lax.fori_loop(0, CH // LANES, group_body, 0)

    def pass2(rows):
        gk = [gam_v[pl.ds(k * LANES, LANES)] for k in range(DIM // LANES)]
        bk = [bet_v[pl.ds(k * LANES, LANES)] for k in range(DIM // LANES)]

        def group_body(g, _):
            mean = mn_v[pl.ds(g * LANES, LANES)]
            rstd = rs_v[pl.ds(g * LANES, LANES)]
            t0 = g * LANES
            TB = 8
            for l in range(0, LANES, TB):
                bms = [lane_gather(mean, jnp.full((LANES,), l + j,
                                                  dtype=jnp.int32))
                       for j in range(TB)]
                brs = [lane_gather(rstd, jnp.full((LANES,), l + j,
                                                  dtype=jnp.int32))
                       for j in range(TB)]
                for k in range(DIM // LANES):
                    xs = [rows[t0 + l + j, pl.ds(k * LANES, LANES)]
                          for j in range(TB)]
                    ys = [((x - bm) * br) * gk[k] + bk[k]
                          for x, bm, br in zip(xs, bms, brs)]
                    for j in range(TB):
                        ybuf[t0 + l + j, pl.ds(k * LANES, LANES)] = ys[j]
            return 0

        lax.fori_loop(0, CH // LANES, group_body, 0)

    fire_gather(0, rows0, gsem0)

    def pair_body(p, _):
        c0 = p * 2
        c1 = c0 + 1
        wait_gather(rows0, gsem0)
        fire_gather(c1, rows1, gsem1)
        pass1(c0, rows0)

        @pl.when(p > 0)
        def _():
            wait_store(osem)

        pass2(rows0)
        fire_store(c0, osem)

        wait_gather(rows1, gsem1)

        @pl.when(p < NCH // 2 - 1)
        def _():
            fire_gather(c0 + 2, rows0, gsem0)

        pass1(c1, rows1)
        wait_store(osem)
        pass2(rows1)
        fire_store(c1, osem)
        return 0

    lax.fori_loop(0, NCH // 2, pair_body, 0)
    wait_store(osem)


@functools.partial(
    pl.kernel,
    mesh=plsc.VectorSubcoreMesh(core_axis_name="c", subcore_axis_name="s"),
    out_type=jax.ShapeDtypeStruct((NTOK, 128), jnp.float32),
    compiler_params=pltpu.CompilerParams(use_tc_tiling_on_sc=False,
                                        needs_layout_passes=False),
    scratch_types=[
        pltpu.VMEM((IDROWS, 128), jnp.int32),
        pltpu.VMEM((CH, 128), jnp.float32),
        pltpu.VMEM((CH, 128), jnp.float32),
        pltpu.VMEM((CH, DIM), jnp.float32),
        pltpu.VMEM((SEQ + 1, DIM), jnp.float32),
        pltpu.VMEM((DIM,), jnp.float32),
        pltpu.VMEM((DIM,), jnp.float32),
        pltpu.VMEM((CH,), jnp.float32),
        pltpu.VMEM((CH,), jnp.float32),
        pltpu.SemaphoreType.DMA,
        pltpu.SemaphoreType.DMA,
        pltpu.SemaphoreType.DMA,
    ],
)
def _sc_call(ids_hbm, tok_hbm, comb_hbm, gam_hbm, bet_hbm, out_hbm,
             ids_v, rows0, rows1, ybuf,
             comb_v, gam_v, bet_v, mn_v, rs_v, gsem0, gsem1, osem):
    _sc_embed_ln(ids_hbm, tok_hbm, comb_hbm, gam_hbm, bet_hbm, out_hbm,
                 ids_v, rows0, rows1, ybuf,
                 comb_v, gam_v, bet_v, mn_v, rs_v, gsem0, gsem1, osem)


def kernel(input_ids, tok_table, pos_table, type_table, gamma, beta):
    b, seq = input_ids.shape
    ids2d = input_ids.reshape(-1).reshape(NTOK // 128, 128)
    combined = jnp.concatenate(
        [(pos_table[0] + type_table[0])[None, :],
         pos_table[1:1 + seq] + type_table[1][None, :]], axis=0)
    tokp = jnp.pad(tok_table, ((0, 0), (0, 64)))
    out = _sc_call(ids2d, tokp, combined, gamma, beta)
    return out[:, :DIM].reshape(b, seq, DIM)
